# Initial kernel scaffold; baseline (speedup 1.0000x reference)
import jax
import jax.numpy as jnp
from jax.experimental import pallas as pl


def kernel(gnnfeats, gnnpos, params, batch_idx):
    raise NotImplementedError("write your pallas kernel here")


# speedup vs baseline: 2.1088x; 2.1088x over previous
import jax
import jax.numpy as jnp
from jax.experimental import pallas as pl
from jax.experimental.pallas import tpu as pltpu

_N = 20
_K = 16
_E = _N * _K
_EPS = 1e-5


def _bn_rows(x):
    m = jnp.mean(x, axis=0, keepdims=True)
    c = x - m
    v = jnp.mean(c * c, axis=0, keepdims=True)
    return c / jnp.sqrt(v + _EPS)


def _ln_row(x):
    m = jnp.mean(x, axis=1, keepdims=True)
    c = x - m
    v = jnp.mean(c * c, axis=1, keepdims=True)
    return c / jnp.sqrt(v + _EPS)


def _elu(x):
    return jnp.where(x > 0, x, jnp.exp(jnp.minimum(x, 0.0)) - 1.0)


def _norm_relu(z, m, v):
    return jnp.maximum((z - m) / jnp.sqrt(v + _EPS), 0.0)


def _knn_pairs(pts, fts, w1):
    cols = [jnp.sum((pts - pts[j:j + 1, :]) ** 2, axis=1, keepdims=True)
            for j in range(_N)]
    d = jnp.concatenate(cols, axis=1)
    col = jax.lax.broadcasted_iota(jnp.int32, (_N, _N), 1)
    row = jax.lax.broadcasted_iota(jnp.int32, (_N, _N), 0)
    d = d + jnp.where(col == row, 1e10, 0.0)
    cnt = jnp.zeros((_N, _N), jnp.float32)
    for jp in range(_N):
        dj = d[:, jp:jp + 1]
        beats = (dj < d) | ((dj == d) & (jp < col))
        cnt = cnt + beats.astype(jnp.float32)
    tmp = jnp.concatenate(
        [jnp.concatenate([fts, fts[j:j + 1, :] - fts], axis=1)
         for j in range(_N)], axis=0)
    z1 = jnp.dot(tmp, w1, preferred_element_type=jnp.float32)
    return cnt, z1


def _start_kernel(gf_ref, pos_ref, w1_ref, fts_o, cnt_o, z1_o):
    fts = _bn_rows(gf_ref[...])
    cnt, z1 = _knn_pairs(pos_ref[...], fts, w1_ref[...])
    fts_o[...] = fts
    cnt_o[...] = cnt
    z1_o[...] = z1


def _mid_kernel(h_ref, w_ref, z_o):
    z_o[...] = jnp.dot(h_ref[...], w_ref[...],
                       preferred_element_type=jnp.float32)


def _aggr(h):
    rows = [jnp.max(h[i * _K:(i + 1) * _K], axis=0, keepdims=True)
            for i in range(_N)]
    return jnp.concatenate(rows, axis=0)


def _finish_kernel(h_ref, fts_ref, wsk_ref, w1_ref, fts_o, cnt_o, z1_o):
    fts = fts_ref[...]
    skip = _bn_rows(jnp.dot(fts, wsk_ref[...],
                            preferred_element_type=jnp.float32))
    new_fts = jnp.maximum(_aggr(h_ref[...]) + skip, 0.0)
    fts_o[...] = new_fts
    cnt, z1 = _knn_pairs(new_fts, new_fts, w1_ref[...])
    cnt_o[...] = cnt
    z1_o[...] = z1


def _last_kernel(h_ref, fts_ref, wsk_ref, fts_o):
    skip = _bn_rows(jnp.dot(fts_ref[...], wsk_ref[...],
                            preferred_element_type=jnp.float32))
    fts_o[...] = jnp.maximum(_aggr(h_ref[...]) + skip, 0.0)


def _gather_kernel(perm_ref, z_ref, zc_ref):
    def body(e, carry):
        p = perm_ref[e]
        zc_ref[pl.ds(e, 1), :] = z_ref[pl.ds(p, 1), :]
        return carry
    jax.lax.fori_loop(0, _E, body, 0)


def _ffn_kernel(*refs):
    (gf_ref, fts_ref,
     e0, e1, e2, e3, e4, eb0, eb1, eb2, eb3, eb4,
     t0, t1, t2, t3, t4, tb0, tb1, tb2, tb3, tb4,
     q0, q1, q2, q3, q4, qb0, qb1, qb2, qb3, qb4,
     o_tau, o_ist, o_p4) = refs
    gf = gf_ref[...]
    fts = fts_ref[...]

    pooled = jnp.sum(fts, axis=0, keepdims=True) * (1.0 / float(_N))
    o_tau[...] = 1.0 / (1.0 + jnp.exp(-pooled))

    s_all = (jnp.sum(jnp.sum(gf, axis=1, keepdims=True), axis=0,
                     keepdims=True)
             + jnp.sum(jnp.sum(fts, axis=1, keepdims=True), axis=0,
                       keepdims=True))
    m = s_all * (1.0 / 600.0)
    cg = gf - m
    cf = fts - m
    v = (jnp.sum(jnp.sum(cg * cg, axis=1, keepdims=True), axis=0,
                 keepdims=True)
         + jnp.sum(jnp.sum(cf * cf, axis=1, keepdims=True), axis=0,
                   keepdims=True)) * (1.0 / 600.0)
    scale = 1.0 / jnp.sqrt(v + _EPS)
    w600 = e0[...]
    acc = eb0[...]
    for n in range(_N):
        xn = jnp.concatenate([cg[n:n + 1, :], cf[n:n + 1, :]], axis=1) * scale
        acc = acc + jnp.dot(xn, w600[n], preferred_element_type=jnp.float32)
    h = _ln_row(_elu(acc))
    for wref, bref in ((e1, eb1), (e2, eb2), (e3, eb3)):
        h = _ln_row(_elu(
            jnp.dot(h, wref[...], preferred_element_type=jnp.float32)
            + bref[...]))
    h = _ln_row(_elu(h))
    enc = jnp.dot(h, e4[...], preferred_element_type=jnp.float32) + eb4[...]

    def head(ws, bs, out):
        hh = _ln_row(enc)
        for i in range(4):
            hh = _ln_row(_elu(
                jnp.dot(hh, ws[i][...], preferred_element_type=jnp.float32)
                + bs[i][...]))
        hh = _ln_row(_elu(hh))
        out[...] = (jnp.dot(hh, ws[4][...],
                            preferred_element_type=jnp.float32) + bs[4][...])

    head((t0, t1, t2, t3, t4), (tb0, tb1, tb2, tb3, tb4), o_ist)
    head((q0, q1, q2, q3, q4), (qb0, qb1, qb2, qb3, qb4), o_p4)


def _call(fn, outs, *args):
    res = pl.pallas_call(
        fn, out_shape=tuple(jax.ShapeDtypeStruct(s, jnp.float32)
                            for s in outs))(*args)
    return res[0] if len(outs) == 1 else res


def _compact(z, perm):
    return pl.pallas_call(
        _gather_kernel,
        grid_spec=pltpu.PrefetchScalarGridSpec(num_scalar_prefetch=1),
        out_shape=jax.ShapeDtypeStruct((_E, z.shape[1]), jnp.float32),
    )(perm, z)


def _bn_relu(z):
    m, v = jax.lax.optimization_barrier(
        (jnp.mean(z, axis=0), jnp.var(z, axis=0)))
    return jax.lax.optimization_barrier(
        jax.nn.relu((z - m) / jnp.sqrt(v + _EPS)))


def _perm_of(cnt):
    i_idx = jax.lax.broadcasted_iota(jnp.int32, (_N, _N), 0)
    j_idx = jax.lax.broadcasted_iota(jnp.int32, (_N, _N), 1)
    cnti = cnt.astype(jnp.int32)
    sel = cnti < _K
    e_f = jnp.where(sel, i_idx * _K + cnti, _E).reshape(-1)
    p_f = (j_idx * _N + i_idx).reshape(-1)
    perm = jnp.zeros((_E + 1,), jnp.int32).at[e_f].set(p_f, mode='drop')
    return perm[:_E], sel.astype(jnp.float32)


def kernel(gnnfeats, gnnpos, params, batch_idx):
    del batch_idx
    gf = gnnfeats[0]
    pos = gnnpos[0]
    convs = [params['conv%d' % l] for l in range(3)]
    widths = [64, 128, 20]

    fts, cnt, z = _call(
        _start_kernel, [(_N, 10), (_N, _N), (_N * _N, 64)],
        gf, pos, convs[0]['W1'])
    for l in range(3):
        p = convs[l]
        perm, _ = _perm_of(cnt)
        h = _bn_relu(_compact(z, perm))
        z = _call(_mid_kernel, [(_E, p['W2'].shape[1])], h, p['W2'])
        h = _bn_relu(z)
        z = _call(_mid_kernel, [(_E, p['W3'].shape[1])], h, p['W3'])
        h = _bn_relu(z)
        if l < 2:
            nxt = convs[l + 1]
            fts, cnt, z = _call(
                _finish_kernel,
                [(_N, widths[l]), (_N, _N),
                 (_N * _N, nxt['W1'].shape[1])],
                h, fts, p['Wskip'], nxt['W1'])
        else:
            fts = _call(_last_kernel, [(_N, widths[l])],
                        h, fts, p['Wskip'])

    enc = params['nn_encode']
    args = [gf, fts, enc[0][0].reshape(_N, 30, 128)]
    args += [enc[i][0] for i in range(1, 5)]
    args += [enc[i][1].reshape(1, -1) for i in range(5)]
    for nm in ('nn_pred_istau', 'nn_pred_p4'):
        ps = params[nm]
        args += [ps[i][0] for i in range(5)]
        args += [ps[i][1].reshape(1, -1) for i in range(5)]
    o_tau, o_ist, o_p4 = _call(_ffn_kernel, [(1, _N), (1, 1), (1, 4)], *args)
    return (o_tau, o_ist.reshape(1), o_p4.reshape(4))
